# baseline (speedup 1.0000x reference)
import functools

import jax
import jax.numpy as jnp
from jax import lax
from jax.experimental import pallas as pl
from jax.experimental.pallas import tpu as pltpu
from jax.experimental.pallas import tpu_sc as plsc

D = 128
DA = 144
DEG_COL = D
NC, NS = 2, 16
NW = NC * NS
CHUNK = 128
NBUF = 2


def _sc_aggregate(x_aug, edges_blk, n_nodes, e_per_tile):
    n_chunks = e_per_tile // CHUNK
    assert n_chunks % NBUF == 0
    n_acc = ((n_nodes + 1 + NS * CHUNK - 1) // (NS * CHUNK)) * (NS * CHUNK)
    rows_per_tile = n_acc // NS
    n_pieces = rows_per_tile // CHUNK
    mesh = plsc.VectorSubcoreMesh(core_axis_name="c", subcore_axis_name="s")

    @functools.partial(
        pl.kernel,
        out_type=jax.ShapeDtypeStruct((NC, n_acc, DA), jnp.float32),
        mesh=mesh,
        scratch_types=[
            pltpu.VMEM_SHARED((n_acc, DA), jnp.float32),
            pltpu.VMEM((2, CHUNK), jnp.int32),
            pltpu.VMEM((2, CHUNK), jnp.int32),
            pltpu.VMEM((CHUNK, DA), jnp.float32),
            pltpu.VMEM((CHUNK, DA), jnp.float32),
            pltpu.SemaphoreType.DMA,
            pltpu.SemaphoreType.DMA,
        ],
        compiler_params=pltpu.CompilerParams(use_tc_tiling_on_sc=False),
    )
    def agg(x_hbm, edges_hbm, out_hbm, acc, eidx0, eidx1, rows0, rows1, sem0, sem1):
        c = lax.axis_index("c")
        s = lax.axis_index("s")
        slots = ((eidx0, rows0, sem0), (eidx1, rows1, sem1))

        zv = jnp.zeros((16,), jnp.float32)

        def fill(i, carry):
            for j in range(DA // 16):
                rows0[i, pl.ds(j * 16, 16)] = zv
            return carry

        lax.fori_loop(0, CHUNK, fill, 0)
        r0 = s * rows_per_tile
        for k in range(n_pieces):
            pltpu.sync_copy(rows0, acc.at[pl.ds(r0 + k * CHUNK, CHUNK)])
        plsc.subcore_barrier()

        wid = c * NS + s
        chunk0 = wid * n_chunks

        for b in range(NBUF):
            eidx, rows, sem = slots[b]
            pltpu.sync_copy(edges_hbm.at[chunk0 + b], eidx)
            pltpu.async_copy(x_hbm.at[eidx.at[0]], rows, sem)

        def outer(t, carry):
            for b in range(NBUF):
                g = t * NBUF + b
                eidx, rows, sem = slots[b]
                pltpu.make_async_copy(x_hbm.at[eidx.at[0]], rows, sem).wait()
                pltpu.sync_copy(rows, acc.at[eidx.at[1]], add=True)

                @pl.when(g + NBUF < n_chunks)
                def _():
                    pltpu.sync_copy(edges_hbm.at[chunk0 + g + NBUF], eidx)
                    pltpu.async_copy(x_hbm.at[eidx.at[0]], rows, sem)

            return carry

        lax.fori_loop(0, n_chunks // NBUF, outer, 0)
        plsc.subcore_barrier()

        for k in range(n_pieces):
            pltpu.sync_copy(acc.at[pl.ds(r0 + k * CHUNK, CHUNK)], rows0)
            pltpu.sync_copy(rows0, out_hbm.at[c, pl.ds(r0 + k * CHUNK, CHUNK)])

    return agg(x_aug, edges_blk)


def _tc_head(x, partial, W, b):
    n = x.shape[0]
    R = 1000
    grid = (n // R,)

    def body(x_ref, p_ref, w_ref, b_ref, o_ref):
        xb = x_ref[...]
        p = p_ref[...]
        accb = p[0] + p[1]
        ssum = accb[:, :D]
        deg = accb[:, DEG_COL:DEG_COL + 1]
        mean = ssum / jnp.maximum(deg, 1.0)
        w = w_ref[...]
        h = (
            jnp.dot(xb, w[:D], preferred_element_type=jnp.float32,
                    precision=lax.Precision.HIGHEST)
            + jnp.dot(mean, w[D:], preferred_element_type=jnp.float32,
                      precision=lax.Precision.HIGHEST)
            + b_ref[...]
        )
        h = jnp.maximum(h, 0.0)
        nrm = jnp.sqrt(jnp.sum(h * h, axis=1, keepdims=True))
        o_ref[...] = h / jnp.maximum(nrm, 1e-12)

    return pl.pallas_call(
        body,
        grid=grid,
        in_specs=[
            pl.BlockSpec((R, D), lambda i: (i, 0)),
            pl.BlockSpec((NC, R, DA), lambda i: (0, i, 0)),
            pl.BlockSpec((2 * D, D), lambda i: (0, 0)),
            pl.BlockSpec((1, D), lambda i: (0, 0)),
        ],
        out_specs=pl.BlockSpec((R, D), lambda i: (i, 0)),
        out_shape=jax.ShapeDtypeStruct((n, D), jnp.float32),
    )(x, partial, W, b.reshape(1, D))


def kernel(input_matrix, adjacency_coo_matrix, W, b):
    x = input_matrix
    n = x.shape[0]
    e = adjacency_coo_matrix.shape[1]
    per_tile_chunks = (e + NW * CHUNK - 1) // (NW * CHUNK)
    per_tile_chunks = ((per_tile_chunks + NBUF - 1) // NBUF) * NBUF
    e_per_tile = per_tile_chunks * CHUNK
    e_pad = NW * e_per_tile
    pad = e_pad - e
    src = adjacency_coo_matrix[0].astype(jnp.int32)
    dst = adjacency_coo_matrix[1].astype(jnp.int32)
    src_p = jnp.concatenate([src, jnp.zeros((pad,), jnp.int32)])
    dst_p = jnp.concatenate([dst, jnp.full((pad,), n, jnp.int32)])
    edges_blk = jnp.stack(
        [src_p.reshape(-1, CHUNK), dst_p.reshape(-1, CHUNK)], axis=1)
    ones = jnp.ones((n, 1), x.dtype)
    zpad = jnp.zeros((n, DA - D - 1), x.dtype)
    x_aug = jnp.concatenate([x, ones, zpad], axis=1)
    partial = _sc_aggregate(x_aug, edges_blk, n, e_per_tile)
    return _tc_head(x, partial, W, b)


# speedup vs baseline: 2.2232x; 2.2232x over previous
import functools

import jax
import jax.numpy as jnp
from jax import lax
from jax.experimental import pallas as pl
from jax.experimental.pallas import tpu as pltpu
from jax.experimental.pallas import tpu_sc as plsc

D = 128
DA = 144
DEG_COL = D
NC, NS = 2, 16
NW = NC * NS
CHUNK = 128
NBUF = 2


def _sc_aggregate(x_aug, edges_blk, n_nodes, e_per_tile):
    n_chunks = e_per_tile // CHUNK
    assert n_chunks % NBUF == 0
    n_acc = ((n_nodes + 1 + NS * CHUNK - 1) // (NS * CHUNK)) * (NS * CHUNK)
    rows_per_tile = n_acc // NS
    n_pieces = rows_per_tile // CHUNK
    mesh = plsc.VectorSubcoreMesh(core_axis_name="c", subcore_axis_name="s")

    @functools.partial(
        pl.kernel,
        out_type=jax.ShapeDtypeStruct((NC, n_acc, DA), jnp.float32),
        mesh=mesh,
        scratch_types=[
            pltpu.VMEM_SHARED((n_acc, DA), jnp.float32),
            pltpu.VMEM((2, CHUNK), jnp.int32),
            pltpu.VMEM((2, CHUNK), jnp.int32),
            pltpu.VMEM((CHUNK, DA), jnp.float32),
            pltpu.VMEM((CHUNK, DA), jnp.float32),
            pltpu.SemaphoreType.DMA,
            pltpu.SemaphoreType.DMA,
        ],
        compiler_params=pltpu.CompilerParams(use_tc_tiling_on_sc=False),
    )
    def agg(x_hbm, edges_hbm, out_hbm, acc, eidx0, eidx1, rows0, rows1, sem0, sem1):
        c = lax.axis_index("c")
        s = lax.axis_index("s")
        slots = ((eidx0, rows0, sem0), (eidx1, rows1, sem1))

        zv = jnp.zeros((16,), jnp.float32)

        def fill(i, carry):
            for j in range(DA // 16):
                rows0[i, pl.ds(j * 16, 16)] = zv
            return carry

        lax.fori_loop(0, CHUNK, fill, 0)
        r0 = s * rows_per_tile
        for k in range(n_pieces):
            pltpu.sync_copy(rows0, acc.at[pl.ds(r0 + k * CHUNK, CHUNK)])
        plsc.subcore_barrier()

        wid = c * NS + s
        chunk0 = wid * n_chunks

        for b in range(NBUF):
            eidx, rows, sem = slots[b]
            pltpu.sync_copy(edges_hbm.at[chunk0 + b], eidx)
            pltpu.async_copy(x_hbm.at[eidx.at[0]], rows, sem)

        def outer(t, carry):
            for b in range(NBUF):
                g = t * NBUF + b
                eidx, rows, sem = slots[b]
                pltpu.make_async_copy(x_hbm.at[eidx.at[0]], rows, sem).wait()
                pltpu.sync_copy(rows, acc.at[eidx.at[1]], add=True)

                @pl.when(g + NBUF < n_chunks)
                def _():
                    pltpu.sync_copy(edges_hbm.at[chunk0 + g + NBUF], eidx)
                    pltpu.async_copy(x_hbm.at[eidx.at[0]], rows, sem)

            return carry

        lax.fori_loop(0, n_chunks // NBUF, outer, 0)
        plsc.subcore_barrier()

        for k in range(n_pieces):
            pltpu.sync_copy(acc.at[pl.ds(r0 + k * CHUNK, CHUNK)], rows0)
            pltpu.sync_copy(rows0, out_hbm.at[c, pl.ds(r0 + k * CHUNK, CHUNK)])

    return agg(x_aug, edges_blk)


def _tc_head(x, partial, W, b):
    n = x.shape[0]
    R = 1000
    grid = (n // R,)

    def body(x_ref, p_ref, w_ref, b_ref, o_ref):
        xb = x_ref[...]
        p = p_ref[...]
        accb = p[0] + p[1]
        ssum = accb[:, :D]
        deg = accb[:, DEG_COL:DEG_COL + 1]
        mean = ssum / jnp.maximum(deg, 1.0)
        w = w_ref[...]
        h = (
            jnp.dot(xb, w[:D], preferred_element_type=jnp.float32,
                    precision=lax.Precision.HIGHEST)
            + jnp.dot(mean, w[D:], preferred_element_type=jnp.float32,
                      precision=lax.Precision.HIGHEST)
            + b_ref[...]
        )
        h = jnp.maximum(h, 0.0)
        nrm = jnp.sqrt(jnp.sum(h * h, axis=1, keepdims=True))
        o_ref[...] = h / jnp.maximum(nrm, 1e-12)

    return pl.pallas_call(
        body,
        grid=grid,
        in_specs=[
            pl.BlockSpec((R, D), lambda i: (i, 0)),
            pl.BlockSpec((NC, R, DA), lambda i: (0, i, 0)),
            pl.BlockSpec((2 * D, D), lambda i: (0, 0)),
            pl.BlockSpec((1, D), lambda i: (0, 0)),
        ],
        out_specs=pl.BlockSpec((R, D), lambda i: (i, 0)),
        out_shape=jax.ShapeDtypeStruct((n, D), jnp.float32),
    )(x, partial, W, b.reshape(1, D))


def kernel(input_matrix, adjacency_coo_matrix, W, b):
    x = input_matrix
    n = x.shape[0]
    e = adjacency_coo_matrix.shape[1]
    per_tile_chunks = (e + NW * CHUNK - 1) // (NW * CHUNK)
    per_tile_chunks = ((per_tile_chunks + NBUF - 1) // NBUF) * NBUF
    e_per_tile = per_tile_chunks * CHUNK
    e_pad = NW * e_per_tile
    pad = e_pad - e
    src = adjacency_coo_matrix[0].astype(jnp.int32)
    dst = adjacency_coo_matrix[1].astype(jnp.int32)
    n_acc = ((n + 1 + NS * CHUNK - 1) // (NS * CHUNK)) * (NS * CHUNK)
    pad_i = jnp.arange(pad, dtype=jnp.int32)
    src_p = jnp.concatenate([src, pad_i % jnp.int32(n)])
    dst_p = jnp.concatenate([dst, n + pad_i % jnp.int32(n_acc - n)])
    edges_blk = jnp.stack(
        [src_p.reshape(-1, CHUNK), dst_p.reshape(-1, CHUNK)], axis=1)
    ones = jnp.ones((n, 1), x.dtype)
    zpad = jnp.zeros((n, DA - D - 1), x.dtype)
    x_aug = jnp.concatenate([x, ones, zpad], axis=1)
    partial = _sc_aggregate(x_aug, edges_blk, n, e_per_tile)
    return _tc_head(x, partial, W, b)


# speedup vs baseline: 2.5117x; 1.1298x over previous
import functools

import jax
import jax.numpy as jnp
from jax import lax
from jax.experimental import pallas as pl
from jax.experimental.pallas import tpu as pltpu
from jax.experimental.pallas import tpu_sc as plsc

D = 128
DA = 160
DEG_COL = D
NC, NS = 2, 16
NW = NC * NS
CHUNK = 128
NBUF = 4


def _sc_aggregate(x_aug, edges_blk, n_nodes, e_per_tile):
    n_chunks = e_per_tile // CHUNK
    assert n_chunks % NBUF == 0
    n_acc = ((n_nodes + 1 + NS * CHUNK - 1) // (NS * CHUNK)) * (NS * CHUNK)
    rows_per_tile = n_acc // NS
    n_pieces = rows_per_tile // CHUNK
    mesh = plsc.VectorSubcoreMesh(core_axis_name="c", subcore_axis_name="s")

    @functools.partial(
        pl.kernel,
        out_type=jax.ShapeDtypeStruct((NC, n_acc, DA), jnp.bfloat16),
        mesh=mesh,
        scratch_types=(
            [pltpu.VMEM_SHARED((n_acc, DA), jnp.bfloat16)]
            + [pltpu.VMEM((2, CHUNK), jnp.int32) for _ in range(NBUF)]
            + [pltpu.VMEM((CHUNK, DA), jnp.bfloat16) for _ in range(NBUF)]
            + [pltpu.SemaphoreType.DMA for _ in range(NBUF)]
        ),
        compiler_params=pltpu.CompilerParams(use_tc_tiling_on_sc=False),
    )
    def agg(x_hbm, edges_hbm, out_hbm, acc, *bufs):
        eidxs = bufs[:NBUF]
        rows = bufs[NBUF:2 * NBUF]
        sems = bufs[2 * NBUF:3 * NBUF]
        c = lax.axis_index("c")
        s = lax.axis_index("s")

        zv = jnp.zeros((32,), jnp.bfloat16)

        def fill(i, carry):
            for j in range(DA // 32):
                rows[0][i, pl.ds(j * 32, 32)] = zv
            return carry

        lax.fori_loop(0, CHUNK, fill, 0)
        r0 = s * rows_per_tile
        for k in range(n_pieces):
            pltpu.sync_copy(rows[0], acc.at[pl.ds(r0 + k * CHUNK, CHUNK)])
        plsc.subcore_barrier()

        wid = c * NS + s
        chunk0 = wid * n_chunks

        for b in range(NBUF):
            pltpu.sync_copy(edges_hbm.at[chunk0 + b], eidxs[b])
            pltpu.async_copy(x_hbm.at[eidxs[b].at[0]], rows[b], sems[b])

        def outer(t, carry):
            for b in range(NBUF):
                g = t * NBUF + b
                eidx, row, sem = eidxs[b], rows[b], sems[b]
                pltpu.make_async_copy(x_hbm.at[eidx.at[0]], row, sem).wait()
                pltpu.sync_copy(row, acc.at[eidx.at[1]], add=True)

                @pl.when(g + NBUF < n_chunks)
                def _():
                    pltpu.sync_copy(edges_hbm.at[chunk0 + g + NBUF], eidx)
                    pltpu.async_copy(x_hbm.at[eidx.at[0]], row, sem)

            return carry

        lax.fori_loop(0, n_chunks // NBUF, outer, 0)
        plsc.subcore_barrier()

        for k in range(n_pieces):
            pltpu.sync_copy(acc.at[pl.ds(r0 + k * CHUNK, CHUNK)], rows[0])
            pltpu.sync_copy(rows[0], out_hbm.at[c, pl.ds(r0 + k * CHUNK, CHUNK)])

    return agg(x_aug, edges_blk)


def _tc_head(x, partial, W, b):
    n = x.shape[0]
    R = 2000
    grid = (n // R,)

    def body(x_ref, p_ref, w_ref, b_ref, o_ref):
        xb = x_ref[...]
        p = p_ref[...].astype(jnp.float32)
        accb = p[0] + p[1]
        ssum = accb[:, :D]
        deg = accb[:, DEG_COL:DEG_COL + 1]
        mean = ssum / jnp.maximum(deg, 1.0)
        w = w_ref[...]
        h = (
            jnp.dot(xb, w[:D], preferred_element_type=jnp.float32,
                    precision=lax.Precision.HIGHEST)
            + jnp.dot(mean, w[D:], preferred_element_type=jnp.float32,
                      precision=lax.Precision.HIGHEST)
            + b_ref[...]
        )
        h = jnp.maximum(h, 0.0)
        nrm = jnp.sqrt(jnp.sum(h * h, axis=1, keepdims=True))
        o_ref[...] = h / jnp.maximum(nrm, 1e-12)

    return pl.pallas_call(
        body,
        grid=grid,
        in_specs=[
            pl.BlockSpec((R, D), lambda i: (i, 0)),
            pl.BlockSpec((NC, R, DA), lambda i: (0, i, 0)),
            pl.BlockSpec((2 * D, D), lambda i: (0, 0)),
            pl.BlockSpec((1, D), lambda i: (0, 0)),
        ],
        out_specs=pl.BlockSpec((R, D), lambda i: (i, 0)),
        out_shape=jax.ShapeDtypeStruct((n, D), jnp.float32),
    )(x, partial, W, b.reshape(1, D))


def kernel(input_matrix, adjacency_coo_matrix, W, b):
    x = input_matrix
    n = x.shape[0]
    e = adjacency_coo_matrix.shape[1]
    per_tile_chunks = (e + NW * CHUNK - 1) // (NW * CHUNK)
    per_tile_chunks = ((per_tile_chunks + NBUF - 1) // NBUF) * NBUF
    e_per_tile = per_tile_chunks * CHUNK
    e_pad = NW * e_per_tile
    pad = e_pad - e
    src = adjacency_coo_matrix[0].astype(jnp.int32)
    dst = adjacency_coo_matrix[1].astype(jnp.int32)
    n_acc = ((n + 1 + NS * CHUNK - 1) // (NS * CHUNK)) * (NS * CHUNK)
    pad_i = jnp.arange(pad, dtype=jnp.int32)
    src_p = jnp.concatenate([src, pad_i % jnp.int32(n)])
    dst_p = jnp.concatenate([dst, n + pad_i % jnp.int32(n_acc - n)])
    edges_blk = jnp.stack(
        [src_p.reshape(-1, CHUNK), dst_p.reshape(-1, CHUNK)], axis=1)
    xb = x.astype(jnp.bfloat16)
    ones = jnp.ones((n, 1), jnp.bfloat16)
    zpad = jnp.zeros((n, DA - D - 1), jnp.bfloat16)
    x_aug = jnp.concatenate([xb, ones, zpad], axis=1)
    partial = _sc_aggregate(x_aug, edges_blk, n, e_per_tile)
    return _tc_head(x, partial, W, b)
